# baseline (speedup 1.0000x reference)
import functools

import jax
import jax.numpy as jnp
from jax import lax
from jax.experimental import pallas as pl
from jax.experimental.pallas import tpu as pltpu
from jax.experimental.pallas import tpu_sc as plsc

B, L, D = 16, 576, 384
C = 1024
QK_DIM = 384
HEADS = 6
N = B * L
BLK = 1024

NC, NS, LN = 2, 16, 16
CB = 3
NCG = 6
NQ = 4
NE = 8
TPQ = N // NQ
TPE = N // NE
RCH = 128
NRCH = TPQ // RCH
BKR = N // 128


def _assign_kernel(x_ref, means_ref, xn0_ref, xn1_ref, xn2_ref, bkt_ref):
    x = x_ref[...]
    nrm = jnp.sqrt(jnp.sum(x * x, axis=-1, keepdims=True))
    xn = x / jnp.maximum(nrm, 1e-12)
    xn0_ref[...] = xn[:, 0:128]
    xn1_ref[...] = xn[:, 128:256]
    xn2_ref[...] = xn[:, 256:384]
    dists = jax.lax.dot_general(
        xn, means_ref[...], (((1,), (1,)), ((), ())),
        preferred_element_type=jnp.float32)
    bkt_ref[...] = jnp.argmax(dists, axis=-1).astype(jnp.int32).reshape(8, 128)


_GDN = jax.lax.GatherDimensionNumbers(
    offset_dims=(), collapsed_slice_dims=(0,), start_index_map=(0,))


def _bcast_lane(v, j):
    idx = jnp.full((LN, 1), j, jnp.int32)
    return jax.lax.gather(
        v, idx, _GDN, (1,),
        mode=jax.lax.GatherScatterMode.PROMISE_IN_BOUNDS)


def _sc_scatter(xn0_hbm, xn1_hbm, xn2_hbm, bkt_hbm,
                s0, s1, s2, s3, s4, s5, counts_out,
                bkt_v, rows_a, rows_b, acc, cacc, sem_a, sem_b):
    c = lax.axis_index("c")
    s = lax.axis_index("s")
    iotas = [jax.lax.iota(jnp.int32, LN) + kk * LN for kk in range(4)]
    zeros16 = jnp.zeros((LN,), jnp.float32)
    ones16 = jnp.ones((LN,), jnp.float32)
    sums_out = [s0, s1, s2, s3, s4, s5]

    pltpu.sync_copy(bkt_hbm, bkt_v)

    wid = c * NS + s

    @pl.when(wid < NCG * NQ)
    def _col_tile():
        cg = wid // NQ
        q = wid % NQ
        cb = cg // 2
        lo = (cg % 2) * 64

        def zbody(r, carry):
            for kk in range(8):
                acc[r, pl.ds(kk * LN, LN)] = zeros16
            return carry

        lax.fori_loop(0, C // 2, zbody, 0)

        xn_hbm = [xn0_hbm, xn1_hbm, xn2_hbm]
        bufs = [rows_a, rows_b]
        sems = [sem_a, sem_b]

        def start(ch, par):
            t0 = pl.multiple_of(q * TPQ + ch * RCH, 128)
            for cbi in range(CB):
                @pl.when(cb == cbi)
                def _stage(cbi=cbi, t0=t0, par=par):
                    pltpu.async_copy(
                        xn_hbm[cbi].at[pl.ds(t0, RCH)], bufs[par], sems[par])

        def wait(par):
            pltpu.make_async_copy(
                xn_hbm[0].at[pl.ds(0, RCH)], bufs[par], sems[par]).wait()

        def compute(ch, par):
            r0 = q * (BKR // NQ) + ch
            rows_v = bufs[par]
            for k in range(8):
                bv = bkt_v[r0, pl.ds(k * LN, LN)]
                rb = k * LN
                for j in range(LN):
                    bj = _bcast_lane(bv, j)
                    row = bj & 511
                    colb = (bj & 512) >> 3
                    vals = [rows_v[rb + j, pl.ds(lo + kk * LN, LN)]
                            for kk in range(4)]
                    for kk in range(4):
                        plsc.addupdate_scatter(
                            acc, [row, colb | iotas[kk]], vals[kk])

        start(0, 0)

        def pair(p, carry):
            ch0 = p * 2
            start(ch0 + 1, 1)
            wait(0)
            compute(ch0, 0)

            @pl.when(ch0 + 2 < NRCH)
            def _pre():
                start(ch0 + 2, 0)

            wait(1)
            compute(ch0 + 1, 1)
            return carry

        lax.fori_loop(0, NRCH // 2, pair, 0)
        for cgi in range(NCG):
            @pl.when(cg == cgi)
            def _flush(cgi=cgi):
                pltpu.sync_copy(acc, sums_out[cgi].at[q])

    @pl.when(wid >= NCG * NQ)
    def _cnt_tile():
        e = wid - NCG * NQ

        def czbody(r, carry):
            cacc[pl.ds(r * LN, LN)] = zeros16
            return carry

        lax.fori_loop(0, C, czbody, 0)

        def cbody(rr, carry):
            for k in range(8):
                bv = bkt_v[e * (BKR // NE) + rr, pl.ds(k * LN, LN)]
                for j in range(LN):
                    bj = _bcast_lane(bv, j)
                    cidx = (bj << 4) | iotas[0]
                    plsc.addupdate_scatter(cacc, [cidx], ones16)
            return carry

        lax.fori_loop(0, BKR // NE, cbody, 0)
        pltpu.sync_copy(cacc, counts_out.at[e])


_sc_scatter_call = functools.partial(
    pl.kernel,
    out_type=(
        [jax.ShapeDtypeStruct((NQ, C // 2, 128), jnp.float32)
         for _ in range(NCG)]
        + [jax.ShapeDtypeStruct((NE, C * LN), jnp.float32)]
    ),
    mesh=plsc.VectorSubcoreMesh(core_axis_name="c", subcore_axis_name="s"),
    compiler_params=pltpu.CompilerParams(needs_layout_passes=False),
    scratch_types=[
        pltpu.VMEM((BKR, 128), jnp.int32),
        pltpu.VMEM((RCH, 128), jnp.float32),
        pltpu.VMEM((RCH, 128), jnp.float32),
        pltpu.VMEM((C // 2, 128), jnp.float32),
        pltpu.VMEM((C * LN,), jnp.float32),
        pltpu.SemaphoreType.DMA,
        pltpu.SemaphoreType.DMA,
    ],
)(_sc_scatter)


def _finalize_kernel(s0, s1, s2, s3, s4, s5, counts_ref, means_ref,
                     wk_ref, wv_ref, xg_ref, k_ref, v_ref):
    groups = []
    for sref in (s0, s1, s2, s3, s4, s5):
        p = (sref[0] + sref[1]) + (sref[2] + sref[3])
        groups.append(jnp.concatenate([p[:, :64], p[:, 64:]], axis=0))
    s = jnp.concatenate(groups, axis=-1)
    sn = s / jnp.maximum(jnp.sqrt(jnp.sum(s * s, axis=-1, keepdims=True)), 1e-12)
    cnt = counts_ref[0, :, 0:1]
    for e in range(1, NE):
        cnt = cnt + counts_ref[e, :, 0:1]
    xg = jnp.where(cnt == 0.0, means_ref[...], sn)
    xg_ref[...] = xg
    kf = jax.lax.dot_general(
        xg, wk_ref[...], (((1,), (1,)), ((), ())),
        preferred_element_type=jnp.float32)
    vf = jax.lax.dot_general(
        xg, wv_ref[...], (((1,), (1,)), ((), ())),
        preferred_element_type=jnp.float32)
    dh = QK_DIM // HEADS
    for h in range(HEADS):
        k_ref[h] = kf[:, h * dh:(h + 1) * dh]
        v_ref[h] = vf[:, h * dh:(h + 1) * dh]


def kernel(normed_x, x_means, W_k, W_v):
    x = normed_x.reshape(N, D)
    mn = x_means / jnp.maximum(
        jnp.linalg.norm(x_means, axis=-1, keepdims=True), 1e-12)

    xn0, xn1, xn2, bkt = pl.pallas_call(
        _assign_kernel,
        grid=(N // BLK,),
        in_specs=[
            pl.BlockSpec((BLK, D), lambda i: (i, 0)),
            pl.BlockSpec((C, D), lambda i: (0, 0)),
        ],
        out_specs=[
            pl.BlockSpec((BLK, 128), lambda i: (i, 0)),
            pl.BlockSpec((BLK, 128), lambda i: (i, 0)),
            pl.BlockSpec((BLK, 128), lambda i: (i, 0)),
            pl.BlockSpec((8, 128), lambda i: (i, 0)),
        ],
        out_shape=[
            jax.ShapeDtypeStruct((N, 128), jnp.float32),
            jax.ShapeDtypeStruct((N, 128), jnp.float32),
            jax.ShapeDtypeStruct((N, 128), jnp.float32),
            jax.ShapeDtypeStruct((BKR, 128), jnp.int32),
        ],
    )(x, mn)

    outs = _sc_scatter_call(xn0, xn1, xn2, bkt)
    sums6, counts8 = outs[:NCG], outs[NCG].reshape(NE, C, LN)

    xg, k, v = pl.pallas_call(
        _finalize_kernel,
        out_shape=[
            jax.ShapeDtypeStruct((C, D), jnp.float32),
            jax.ShapeDtypeStruct((HEADS, C, QK_DIM // HEADS), jnp.float32),
            jax.ShapeDtypeStruct((HEADS, C, D // HEADS), jnp.float32),
        ],
    )(*sums6, counts8, mn, W_k, W_v)

    return (k, v, xg)


# speedup vs baseline: 1.0919x; 1.0919x over previous
import functools

import jax
import jax.numpy as jnp
from jax import lax
from jax.experimental import pallas as pl
from jax.experimental.pallas import tpu as pltpu
from jax.experimental.pallas import tpu_sc as plsc

B, L, D = 16, 576, 384
C = 1024
QK_DIM = 384
HEADS = 6
N = B * L
BLK = 1024

NC, NS, LN = 2, 16, 16
CB = 3
NCG = 6
NQ = 4
NE = 8
TPQ = N // NQ
TPE = N // NE
RCH = 256
NRCH = TPQ // RCH
BKR = N // 128


def _assign_kernel(x_ref, means_ref, xn0_ref, xn1_ref, xn2_ref, bkt_ref):
    x = x_ref[...]
    nrm = jnp.sqrt(jnp.sum(x * x, axis=-1, keepdims=True))
    xn = x / jnp.maximum(nrm, 1e-12)
    xn0_ref[...] = xn[:, 0:128]
    xn1_ref[...] = xn[:, 128:256]
    xn2_ref[...] = xn[:, 256:384]
    dists = jax.lax.dot_general(
        xn, means_ref[...], (((1,), (1,)), ((), ())),
        preferred_element_type=jnp.float32)
    bkt_ref[...] = jnp.argmax(dists, axis=-1).astype(jnp.int32).reshape(8, 128)


_GDN = jax.lax.GatherDimensionNumbers(
    offset_dims=(), collapsed_slice_dims=(0,), start_index_map=(0,))


def _bcast_lane(v, j):
    idx = jnp.full((LN, 1), j, jnp.int32)
    return jax.lax.gather(
        v, idx, _GDN, (1,),
        mode=jax.lax.GatherScatterMode.PROMISE_IN_BOUNDS)


def _sc_scatter(xn0_hbm, xn1_hbm, xn2_hbm, bkt_hbm,
                s0, s1, s2, s3, s4, s5, counts_out,
                bkt_v, rows_v, acc, cacc):
    c = lax.axis_index("c")
    s = lax.axis_index("s")
    iotas = [jax.lax.iota(jnp.int32, LN) + kk * LN for kk in range(4)]
    zeros16 = jnp.zeros((LN,), jnp.float32)
    ones16 = jnp.ones((LN,), jnp.float32)
    sums_out = [s0, s1, s2, s3, s4, s5]

    pltpu.sync_copy(bkt_hbm, bkt_v)

    wid = c * NS + s

    @pl.when(wid < NCG * NQ)
    def _col_tile():
        cg = wid // NQ
        q = wid % NQ
        cb = cg // 2
        lo = (cg % 2) * 64

        def zbody(r, carry):
            for kk in range(8):
                acc[r, pl.ds(kk * LN, LN)] = zeros16
            return carry

        lax.fori_loop(0, C // 2, zbody, 0)

        xn_hbm = [xn0_hbm, xn1_hbm, xn2_hbm]

        def chunk(ch, carry):
            t0 = pl.multiple_of(q * TPQ + ch * RCH, 128)
            for cbi in range(CB):
                @pl.when(cb == cbi)
                def _stage(cbi=cbi, t0=t0):
                    pltpu.sync_copy(xn_hbm[cbi].at[pl.ds(t0, RCH)], rows_v)
            r0 = q * (BKR // NQ) + ch * (RCH // 128)

            def body(rr, carry2):
                for k in range(8):
                    bv = bkt_v[r0 + rr, pl.ds(k * LN, LN)]
                    rb = rr * 128 + k * LN
                    for j in range(LN):
                        bj = _bcast_lane(bv, j)
                        row = bj & 511
                        colb = (bj & 512) >> 3
                        vals = [rows_v[rb + j, pl.ds(lo + kk * LN, LN)]
                                for kk in range(4)]
                        for kk in range(4):
                            plsc.addupdate_scatter(
                                acc, [row, colb | iotas[kk]], vals[kk])
                return carry2

            lax.fori_loop(0, RCH // 128, body, 0)
            return carry

        lax.fori_loop(0, NRCH, chunk, 0)
        for cgi in range(NCG):
            @pl.when(cg == cgi)
            def _flush(cgi=cgi):
                pltpu.sync_copy(acc, sums_out[cgi].at[q])

    @pl.when(wid >= NCG * NQ)
    def _cnt_tile():
        e = wid - NCG * NQ

        def czbody(r, carry):
            cacc[pl.ds(r * LN, LN)] = zeros16
            return carry

        lax.fori_loop(0, C, czbody, 0)

        def cbody(rr, carry):
            for k in range(8):
                bv = bkt_v[e * (BKR // NE) + rr, pl.ds(k * LN, LN)]
                for j in range(LN):
                    bj = _bcast_lane(bv, j)
                    cidx = (bj << 4) | iotas[0]
                    plsc.addupdate_scatter(cacc, [cidx], ones16)
            return carry

        lax.fori_loop(0, BKR // NE, cbody, 0)
        pltpu.sync_copy(cacc, counts_out.at[e])


_sc_scatter_call = functools.partial(
    pl.kernel,
    out_type=(
        [jax.ShapeDtypeStruct((NQ, C // 2, 128), jnp.float32)
         for _ in range(NCG)]
        + [jax.ShapeDtypeStruct((NE, C * LN), jnp.float32)]
    ),
    mesh=plsc.VectorSubcoreMesh(core_axis_name="c", subcore_axis_name="s"),
    compiler_params=pltpu.CompilerParams(needs_layout_passes=False),
    scratch_types=[
        pltpu.VMEM((BKR, 128), jnp.int32),
        pltpu.VMEM((RCH, 128), jnp.float32),
        pltpu.VMEM((C // 2, 128), jnp.float32),
        pltpu.VMEM((C * LN,), jnp.float32),
    ],
)(_sc_scatter)


def _finalize_kernel(s0, s1, s2, s3, s4, s5, counts_ref, means_ref,
                     wk_ref, wv_ref, xg_ref, k_ref, v_ref):
    groups = []
    for sref in (s0, s1, s2, s3, s4, s5):
        p = (sref[0] + sref[1]) + (sref[2] + sref[3])
        groups.append(jnp.concatenate([p[:, :64], p[:, 64:]], axis=0))
    s = jnp.concatenate(groups, axis=-1)
    sn = s / jnp.maximum(jnp.sqrt(jnp.sum(s * s, axis=-1, keepdims=True)), 1e-12)
    cnt = counts_ref[0, :, 0:1]
    for e in range(1, NE):
        cnt = cnt + counts_ref[e, :, 0:1]
    xg = jnp.where(cnt == 0.0, means_ref[...], sn)
    xg_ref[...] = xg
    k_ref[...] = jax.lax.dot_general(
        xg, wk_ref[...], (((1,), (1,)), ((), ())),
        preferred_element_type=jnp.float32)
    v_ref[...] = jax.lax.dot_general(
        xg, wv_ref[...], (((1,), (1,)), ((), ())),
        preferred_element_type=jnp.float32)


def kernel(normed_x, x_means, W_k, W_v):
    x = normed_x.reshape(N, D)
    mn = x_means / jnp.maximum(
        jnp.linalg.norm(x_means, axis=-1, keepdims=True), 1e-12)

    xn0, xn1, xn2, bkt = pl.pallas_call(
        _assign_kernel,
        grid=(N // BLK,),
        in_specs=[
            pl.BlockSpec((BLK, D), lambda i: (i, 0)),
            pl.BlockSpec((C, D), lambda i: (0, 0)),
        ],
        out_specs=[
            pl.BlockSpec((BLK, 128), lambda i: (i, 0)),
            pl.BlockSpec((BLK, 128), lambda i: (i, 0)),
            pl.BlockSpec((BLK, 128), lambda i: (i, 0)),
            pl.BlockSpec((8, 128), lambda i: (i, 0)),
        ],
        out_shape=[
            jax.ShapeDtypeStruct((N, 128), jnp.float32),
            jax.ShapeDtypeStruct((N, 128), jnp.float32),
            jax.ShapeDtypeStruct((N, 128), jnp.float32),
            jax.ShapeDtypeStruct((BKR, 128), jnp.int32),
        ],
    )(x, mn)

    outs = _sc_scatter_call(xn0, xn1, xn2, bkt)
    sums6, counts8 = outs[:NCG], outs[NCG].reshape(NE, C, LN)

    xg, k, v = pl.pallas_call(
        _finalize_kernel,
        out_shape=[
            jax.ShapeDtypeStruct((C, D), jnp.float32),
            jax.ShapeDtypeStruct((C, QK_DIM), jnp.float32),
            jax.ShapeDtypeStruct((C, D), jnp.float32),
        ],
    )(*sums6, counts8, mn, W_k, W_v)

    k = k.reshape(C, HEADS, QK_DIM // HEADS).transpose(1, 0, 2)
    v = v.reshape(C, HEADS, D // HEADS).transpose(1, 0, 2)
    return (k, v, xg)
